# baseline (speedup 1.0000x reference)
import functools

import jax
import jax.numpy as jnp
from jax import lax
from jax.experimental import pallas as pl
from jax.experimental.pallas import tpu as pltpu
from jax.experimental.pallas import tpu_sc as plsc

N = 4096
D = 768
E = 8
K = 2
U = 16
H = 4 * D

BG = 256
G = (K * N + E * BG) // BG
P = G * BG

NW = 32
PCH = (K * N) // NW
TCH = N // NW
CCH = 64
DW = D // 2


def _gelu_exact(t):
    return 0.5 * t * (1.0 + lax.erf(t * 0.7071067811865476))


def _uncertainty_body(x_ref, wu1_ref, bu1_ref, wu2_ref, bu2_ref, un_ref):
    x = x_ref[...]
    h = lax.dot_general(x, wu1_ref[...], (((1,), (1,)), ((), ())),
                        preferred_element_type=jnp.float32)
    h = _gelu_exact(h + bu1_ref[...])
    u = jnp.sum(h * wu2_ref[...], axis=-1, keepdims=True)
    u = u + bu2_ref[0, 0]
    u = jnp.maximum(u, 0.0) + jnp.log1p(jnp.exp(-jnp.abs(u)))
    un_ref[...] = u / (jnp.mean(u) + 1e-8)


def _router_body(logits_ref, pos_ref, wsl_ref, bexp_ref, aux_ref):
    logits = logits_ref[...]
    m = jnp.max(logits, axis=-1, keepdims=True)
    ex = jnp.exp(logits - m)
    se = jnp.sum(ex, axis=-1, keepdims=True)
    p = ex / se
    iota = lax.broadcasted_iota(jnp.int32, (N, E), 1)
    m1 = jnp.max(p, axis=-1, keepdims=True)
    i1 = jnp.min(jnp.where(p == m1, iota, E), axis=-1, keepdims=True)
    p2 = jnp.where(iota == i1, -jnp.inf, p)
    m2 = jnp.max(p2, axis=-1, keepdims=True)
    i2 = jnp.min(jnp.where(p2 == m2, iota, E), axis=-1, keepdims=True)
    s = m1 + m2
    sel1 = (iota == i1).astype(jnp.float32)
    sel2 = (iota == i2).astype(jnp.float32)
    usage = jnp.mean(p, axis=0, keepdims=True)
    selection = jnp.mean(sel1 + sel2, axis=0, keepdims=True) / K
    lb = E * jnp.sum(usage * selection)
    lz = m + jnp.log(se)
    z = jnp.mean(lz * lz)
    aux_ref[0, 0] = 0.01 * lb + 0.01 * z
    sel = sel1 + sel2
    csum = sel
    k = 1
    while k < N:
        csum = csum + jnp.concatenate(
            [jnp.zeros((k, E), jnp.float32), csum[:N - k, :]], axis=0)
        k *= 2
    rank = (csum - sel).astype(jnp.int32)
    counts = csum[N - 1:N, :].astype(jnp.int32)
    cp = ((counts + (BG - 1)) // BG) * BG
    seg_end = cp
    for k in (1, 2, 4):
        seg_end = seg_end + jnp.concatenate(
            [jnp.zeros((1, k), jnp.int32), seg_end[:, :E - k]], axis=1)
    start = seg_end - cp
    r1 = jnp.sum(jnp.where(iota == i1, rank, 0), axis=-1, keepdims=True)
    r2 = jnp.sum(jnp.where(iota == i2, rank, 0), axis=-1, keepdims=True)
    st1 = jnp.sum(jnp.where(iota == i1, start, 0), axis=-1, keepdims=True)
    st2 = jnp.sum(jnp.where(iota == i2, start, 0), axis=-1, keepdims=True)
    pos_ref[0:N, :] = st1 + r1
    pos_ref[N:2 * N, :] = st2 + r2
    wsl_ref[0:N, :] = m1 / s
    wsl_ref[N:2 * N, :] = m2 / s
    bstart = lax.broadcasted_iota(jnp.int32, (G, E), 0) * BG
    bexp = jnp.sum((bstart >= seg_end).astype(jnp.int32), axis=-1,
                   keepdims=True)
    bexp_ref[...] = jnp.minimum(bexp, E - 1)


DCH = PCH // 2


def _dispatch_body(pos_hbm, x_hbm, xg_hbm, posv, buf0, buf1, sem0, sem1):
    wid = lax.axis_index("s") * 2 + lax.axis_index("c")
    tbase = lax.rem(wid * PCH, N)
    pltpu.sync_copy(pos_hbm.at[wid], posv)
    pltpu.sync_copy(x_hbm.at[pl.ds(tbase, DCH)], buf0)
    cp0 = pltpu.async_copy(buf0, xg_hbm.at[posv.at[0]], sem0)
    pltpu.sync_copy(x_hbm.at[pl.ds(tbase + DCH, DCH)], buf1)
    cp1 = pltpu.async_copy(buf1, xg_hbm.at[posv.at[1]], sem1)
    cp0.wait()
    cp1.wait()


def _ffn_body(be_ref, xg_ref, w1_ref, b1_ref, w2_ref, b2_ref, ys_ref):
    h = lax.dot_general(xg_ref[...], w1_ref[0], (((1,), (1,)), ((), ())),
                        preferred_element_type=jnp.float32)
    h = _gelu_exact(h + b1_ref[0])
    o = lax.dot_general(h.astype(jnp.bfloat16), w2_ref[0],
                        (((1,), (1,)), ((), ())),
                        preferred_element_type=jnp.float32)
    ys_ref[...] = (o + b2_ref[0]).astype(jnp.bfloat16)


def _combine_gather_body(posc_hbm, ys_hbm, y1_hbm, y2_hbm, idxv,
                         bufs, sem):
    wid = lax.axis_index("s") * 2 + lax.axis_index("c")
    tbase = wid * TCH
    pltpu.sync_copy(posc_hbm.at[wid], idxv)
    nch = TCH // CCH
    copies = []
    for c in range(nch):
        copies.append(pltpu.async_copy(
            ys_hbm.at[idxv.at[0, pl.ds(c * CCH, CCH)]], bufs.at[c], sem))
        copies.append(pltpu.async_copy(
            ys_hbm.at[idxv.at[1, pl.ds(c * CCH, CCH)]], bufs.at[nch + c],
            sem))
    for cp in copies:
        cp.wait()
    for c in range(nch):
        pltpu.sync_copy(bufs.at[c], y1_hbm.at[pl.ds(tbase + c * CCH, CCH)])
        pltpu.sync_copy(bufs.at[nch + c],
                        y2_hbm.at[pl.ds(tbase + c * CCH, CCH)])


def _combine_add_body(y1_ref, y2_ref, w1_ref, w2_ref, out_ref):
    out_ref[...] = (y1_ref[...].astype(jnp.float32) * w1_ref[...]
                    + y2_ref[...].astype(jnp.float32) * w2_ref[...])


def _run_router(x, W_u1, b_u1, W_u2, b_u2, W_router):
    un = pl.pallas_call(
        _uncertainty_body,
        out_shape=jax.ShapeDtypeStruct((N, 1), jnp.float32),
        in_specs=[
            pl.BlockSpec((N, D), lambda: (0, 0)),
            pl.BlockSpec((U, D), lambda: (0, 0)),
            pl.BlockSpec((1, U), lambda: (0, 0)),
            pl.BlockSpec((1, U), lambda: (0, 0)),
            pl.BlockSpec(memory_space=pltpu.SMEM),
        ],
        out_specs=pl.BlockSpec((N, 1), lambda: (0, 0)),
    )(x, W_u1, b_u1.reshape(1, U), W_u2, b_u2.reshape(1, 1))
    logits = jnp.concatenate([x, un], axis=-1) @ W_router.T
    return pl.pallas_call(
        _router_body,
        out_shape=(jax.ShapeDtypeStruct((K * N, 1), jnp.int32),
                   jax.ShapeDtypeStruct((K * N, 1), jnp.float32),
                   jax.ShapeDtypeStruct((G, 1), jnp.int32),
                   jax.ShapeDtypeStruct((1, 1), jnp.float32)),
        in_specs=[
            pl.BlockSpec((N, E), lambda: (0, 0)),
        ],
        out_specs=(pl.BlockSpec((K * N, 1), lambda: (0, 0)),
                   pl.BlockSpec((K * N, 1), lambda: (0, 0)),
                   pl.BlockSpec((G, 1), lambda: (0, 0)),
                   pl.BlockSpec(memory_space=pltpu.SMEM)),
    )(logits)


def _run_ffn(bexp_flat, xg_bf, We1b, be1r, We2b, be2r):
    return pl.pallas_call(
        _ffn_body,
        grid_spec=pltpu.PrefetchScalarGridSpec(
            num_scalar_prefetch=1,
            grid=(G,),
            in_specs=[
                pl.BlockSpec((BG, D), lambda g, be: (g, 0)),
                pl.BlockSpec((1, H, D), lambda g, be: (be[g], 0, 0)),
                pl.BlockSpec((1, 1, H), lambda g, be: (be[g], 0, 0)),
                pl.BlockSpec((1, D, H), lambda g, be: (be[g], 0, 0)),
                pl.BlockSpec((1, 1, D), lambda g, be: (be[g], 0, 0)),
            ],
            out_specs=pl.BlockSpec((BG, D), lambda g, be: (g, 0)),
        ),
        out_shape=jax.ShapeDtypeStruct((P, D), jnp.bfloat16),
        compiler_params=pltpu.CompilerParams(
            dimension_semantics=("arbitrary",)),
    )(bexp_flat, xg_bf, We1b, be1r, We2b, be2r)


@jax.jit
def kernel(x, W_u1, b_u1, W_u2, b_u2, W_router, We1, be1, We2, be2):
    pos, wsl, bexp, aux = _run_router(x, W_u1, b_u1, W_u2, b_u2, W_router)

    x_i32 = lax.bitcast_convert_type(
        x.astype(jnp.bfloat16).reshape(N, DW, 2), jnp.int32)

    pf = pos.reshape(K * N)
    pos_d = pf.reshape(NW, 2, DCH)
    posc = jnp.stack([pf[:N].reshape(NW, TCH),
                      pf[N:].reshape(NW, TCH)], axis=1)

    mesh = plsc.VectorSubcoreMesh(core_axis_name="c", subcore_axis_name="s")
    xg = pl.kernel(
        _dispatch_body,
        out_type=jax.ShapeDtypeStruct((P, DW), jnp.int32),
        mesh=mesh,
        compiler_params=pltpu.CompilerParams(needs_layout_passes=False),
        scratch_types=[
            pltpu.VMEM((2, DCH), jnp.int32),
            pltpu.VMEM((DCH, DW), jnp.int32),
            pltpu.VMEM((DCH, DW), jnp.int32),
            pltpu.SemaphoreType.DMA,
            pltpu.SemaphoreType.DMA,
        ],
    )(pos_d, x_i32)

    xg_bf = lax.bitcast_convert_type(xg, jnp.bfloat16).reshape(P, D)
    We1b = We1.astype(jnp.bfloat16)
    We2b = We2.astype(jnp.bfloat16)
    be1r = be1.reshape(E, 1, H)
    be2r = be2.reshape(E, 1, D)

    ys = _run_ffn(bexp.reshape(G), xg_bf, We1b, be1r, We2b, be2r)
    ys_i32 = lax.bitcast_convert_type(ys.reshape(P, DW, 2), jnp.int32)

    y1, y2 = pl.kernel(
        _combine_gather_body,
        out_type=(jax.ShapeDtypeStruct((N, DW), jnp.int32),
                  jax.ShapeDtypeStruct((N, DW), jnp.int32)),
        mesh=plsc.VectorSubcoreMesh(core_axis_name="c",
                                    subcore_axis_name="s"),
        compiler_params=pltpu.CompilerParams(needs_layout_passes=False),
        scratch_types=[
            pltpu.VMEM((2, TCH), jnp.int32),
            pltpu.VMEM((2 * (TCH // CCH), CCH, DW), jnp.int32),
            pltpu.SemaphoreType.DMA,
        ],
    )(posc, ys_i32)

    y1_bf = lax.bitcast_convert_type(y1, jnp.bfloat16).reshape(N, D)
    y2_bf = lax.bitcast_convert_type(y2, jnp.bfloat16).reshape(N, D)

    BA = 1024
    out = pl.pallas_call(
        _combine_add_body,
        grid=(N // BA,),
        in_specs=[
            pl.BlockSpec((BA, D), lambda g: (g, 0)),
            pl.BlockSpec((BA, D), lambda g: (g, 0)),
            pl.BlockSpec((BA, 1), lambda g: (g, 0)),
            pl.BlockSpec((BA, 1), lambda g: (g, 0)),
        ],
        out_specs=pl.BlockSpec((BA, D), lambda g: (g, 0)),
        out_shape=jax.ShapeDtypeStruct((N, D), jnp.float32),
    )(y1_bf, y2_bf, wsl[:N], wsl[N:])

    return out, aux.reshape(())


# speedup vs baseline: 1.5579x; 1.5579x over previous
import functools

import jax
import jax.numpy as jnp
from jax import lax
from jax.experimental import pallas as pl
from jax.experimental.pallas import tpu as pltpu
from jax.experimental.pallas import tpu_sc as plsc

N = 4096
D = 768
E = 8
K = 2
U = 16
H = 4 * D

BG = 256
G = (K * N + E * BG) // BG
P = G * BG

NW = 32
PCH = (K * N) // NW
TCH = N // NW
CCH = 32
DW = D // 2


def _gelu_exact(t):
    return 0.5 * t * (1.0 + lax.erf(t * 0.7071067811865476))


def _uncertainty_body(x_ref, wu1_ref, bu1_ref, wu2_ref, bu2_ref, un_ref):
    x = x_ref[...]
    h = lax.dot_general(x, wu1_ref[...], (((1,), (1,)), ((), ())),
                        preferred_element_type=jnp.float32)
    h = _gelu_exact(h + bu1_ref[...])
    u = jnp.sum(h * wu2_ref[...], axis=-1, keepdims=True)
    u = u + bu2_ref[0, 0]
    u = jnp.maximum(u, 0.0) + jnp.log1p(jnp.exp(-jnp.abs(u)))
    un_ref[...] = u / (jnp.mean(u) + 1e-8)


def _router_body(logits_ref, pos_ref, wsl_ref, bexp_ref, aux_ref):
    logits = logits_ref[...]
    m = jnp.max(logits, axis=-1, keepdims=True)
    ex = jnp.exp(logits - m)
    se = jnp.sum(ex, axis=-1, keepdims=True)
    p = ex / se
    iota = lax.broadcasted_iota(jnp.int32, (N, E), 1)
    m1 = jnp.max(p, axis=-1, keepdims=True)
    i1 = jnp.min(jnp.where(p == m1, iota, E), axis=-1, keepdims=True)
    p2 = jnp.where(iota == i1, -jnp.inf, p)
    m2 = jnp.max(p2, axis=-1, keepdims=True)
    i2 = jnp.min(jnp.where(p2 == m2, iota, E), axis=-1, keepdims=True)
    s = m1 + m2
    sel1 = (iota == i1).astype(jnp.float32)
    sel2 = (iota == i2).astype(jnp.float32)
    usage = jnp.mean(p, axis=0, keepdims=True)
    selection = jnp.mean(sel1 + sel2, axis=0, keepdims=True) / K
    lb = E * jnp.sum(usage * selection)
    lz = m + jnp.log(se)
    z = jnp.mean(lz * lz)
    aux_ref[0, 0] = 0.01 * lb + 0.01 * z
    sel = sel1 + sel2
    csum = sel
    k = 1
    while k < N:
        csum = csum + jnp.concatenate(
            [jnp.zeros((k, E), jnp.float32), csum[:N - k, :]], axis=0)
        k *= 2
    rank = (csum - sel).astype(jnp.int32)
    counts = csum[N - 1:N, :].astype(jnp.int32)
    cp = ((counts + (BG - 1)) // BG) * BG
    seg_end = cp
    for k in (1, 2, 4):
        seg_end = seg_end + jnp.concatenate(
            [jnp.zeros((1, k), jnp.int32), seg_end[:, :E - k]], axis=1)
    start = seg_end - cp
    r1 = jnp.sum(jnp.where(iota == i1, rank, 0), axis=-1, keepdims=True)
    r2 = jnp.sum(jnp.where(iota == i2, rank, 0), axis=-1, keepdims=True)
    st1 = jnp.sum(jnp.where(iota == i1, start, 0), axis=-1, keepdims=True)
    st2 = jnp.sum(jnp.where(iota == i2, start, 0), axis=-1, keepdims=True)
    pos_ref[0:N, :] = st1 + r1
    pos_ref[N:2 * N, :] = st2 + r2
    wsl_ref[0:N, :] = m1 / s
    wsl_ref[N:2 * N, :] = m2 / s
    bstart = lax.broadcasted_iota(jnp.int32, (G, E), 0) * BG
    bexp = jnp.sum((bstart >= seg_end).astype(jnp.int32), axis=-1,
                   keepdims=True)
    bexp_ref[...] = jnp.minimum(bexp, E - 1)


DCH = PCH // 2


def _dispatch_body(pos_hbm, x_hbm, xg_hbm, posv, buf0, buf1, sem0, sem1):
    wid = lax.axis_index("s") * 2 + lax.axis_index("c")
    tbase = lax.rem(wid * PCH, N)
    pltpu.sync_copy(pos_hbm.at[wid], posv)
    pltpu.sync_copy(x_hbm.at[pl.ds(tbase, DCH)], buf0)
    cp0 = pltpu.async_copy(buf0, xg_hbm.at[posv.at[0]], sem0)
    pltpu.sync_copy(x_hbm.at[pl.ds(tbase + DCH, DCH)], buf1)
    cp1 = pltpu.async_copy(buf1, xg_hbm.at[posv.at[1]], sem1)
    cp0.wait()
    cp1.wait()


def _ffn_body(be_ref, xg_ref, w1_ref, b1_ref, w2_ref, b2_ref, ys_ref):
    h = lax.dot_general(xg_ref[...], w1_ref[0], (((1,), (1,)), ((), ())),
                        preferred_element_type=jnp.float32)
    h = _gelu_exact(h + b1_ref[0])
    o = lax.dot_general(h.astype(jnp.bfloat16), w2_ref[0],
                        (((1,), (1,)), ((), ())),
                        preferred_element_type=jnp.float32)
    ys_ref[...] = o + b2_ref[0]


def _combine_gather_body(posc_hbm, ys_hbm, y1_hbm, y2_hbm, idxv,
                         bufs, sem):
    wid = lax.axis_index("s") * 2 + lax.axis_index("c")
    tbase = wid * TCH
    pltpu.sync_copy(posc_hbm.at[wid], idxv)
    nch = TCH // CCH
    copies = [None] * nch

    def drain(c):
        copies[c][0].wait()
        copies[c][1].wait()
        b = 2 * (c % 2)
        pltpu.sync_copy(bufs.at[b], y1_hbm.at[pl.ds(tbase + c * CCH, CCH)])
        pltpu.sync_copy(bufs.at[b + 1],
                        y2_hbm.at[pl.ds(tbase + c * CCH, CCH)])

    for c in range(nch):
        if c >= 2:
            drain(c - 2)
        b = 2 * (c % 2)
        copies[c] = (
            pltpu.async_copy(
                ys_hbm.at[idxv.at[0, pl.ds(c * CCH, CCH)]], bufs.at[b],
                sem),
            pltpu.async_copy(
                ys_hbm.at[idxv.at[1, pl.ds(c * CCH, CCH)]], bufs.at[b + 1],
                sem))
    for c in range(max(nch - 2, 0), nch):
        drain(c)


def _combine_add_body(y1_ref, y2_ref, w1_ref, w2_ref, out_ref):
    out_ref[...] = (y1_ref[...] * w1_ref[...]
                    + y2_ref[...] * w2_ref[...])


def _run_router(x, W_u1, b_u1, W_u2, b_u2, W_router):
    un = pl.pallas_call(
        _uncertainty_body,
        out_shape=jax.ShapeDtypeStruct((N, 1), jnp.float32),
        in_specs=[
            pl.BlockSpec((N, D), lambda: (0, 0)),
            pl.BlockSpec((U, D), lambda: (0, 0)),
            pl.BlockSpec((1, U), lambda: (0, 0)),
            pl.BlockSpec((1, U), lambda: (0, 0)),
            pl.BlockSpec(memory_space=pltpu.SMEM),
        ],
        out_specs=pl.BlockSpec((N, 1), lambda: (0, 0)),
    )(x, W_u1, b_u1.reshape(1, U), W_u2, b_u2.reshape(1, 1))
    logits = jnp.concatenate([x, un], axis=-1) @ W_router.T
    return pl.pallas_call(
        _router_body,
        out_shape=(jax.ShapeDtypeStruct((K * N, 1), jnp.int32),
                   jax.ShapeDtypeStruct((K * N, 1), jnp.float32),
                   jax.ShapeDtypeStruct((G, 1), jnp.int32),
                   jax.ShapeDtypeStruct((1, 1), jnp.float32)),
        in_specs=[
            pl.BlockSpec((N, E), lambda: (0, 0)),
        ],
        out_specs=(pl.BlockSpec((K * N, 1), lambda: (0, 0)),
                   pl.BlockSpec((K * N, 1), lambda: (0, 0)),
                   pl.BlockSpec((G, 1), lambda: (0, 0)),
                   pl.BlockSpec(memory_space=pltpu.SMEM)),
    )(logits)


def _run_ffn(bexp_flat, xg_bf, We1b, be1r, We2b, be2r):
    return pl.pallas_call(
        _ffn_body,
        grid_spec=pltpu.PrefetchScalarGridSpec(
            num_scalar_prefetch=1,
            grid=(G,),
            in_specs=[
                pl.BlockSpec((BG, D), lambda g, be: (g, 0)),
                pl.BlockSpec((1, H, D), lambda g, be: (be[g], 0, 0)),
                pl.BlockSpec((1, 1, H), lambda g, be: (be[g], 0, 0)),
                pl.BlockSpec((1, D, H), lambda g, be: (be[g], 0, 0)),
                pl.BlockSpec((1, 1, D), lambda g, be: (be[g], 0, 0)),
            ],
            out_specs=pl.BlockSpec((BG, D), lambda g, be: (g, 0)),
        ),
        out_shape=jax.ShapeDtypeStruct((P, D), jnp.float32),
        compiler_params=pltpu.CompilerParams(
            dimension_semantics=("arbitrary",)),
    )(bexp_flat, xg_bf, We1b, be1r, We2b, be2r)


@jax.jit
def kernel(x, W_u1, b_u1, W_u2, b_u2, W_router, We1, be1, We2, be2):
    pos, wsl, bexp, aux = _run_router(x, W_u1, b_u1, W_u2, b_u2, W_router)

    x_i32 = lax.bitcast_convert_type(
        x.astype(jnp.bfloat16).reshape(N, DW, 2), jnp.int32)

    pf = pos.reshape(K * N)
    pos_d = pf.reshape(NW, 2, DCH)
    posc = jnp.stack([pf[:N].reshape(NW, TCH),
                      pf[N:].reshape(NW, TCH)], axis=1)

    mesh = plsc.VectorSubcoreMesh(core_axis_name="c", subcore_axis_name="s")
    xg = pl.kernel(
        _dispatch_body,
        out_type=jax.ShapeDtypeStruct((P, DW), jnp.int32),
        mesh=mesh,
        compiler_params=pltpu.CompilerParams(needs_layout_passes=False),
        scratch_types=[
            pltpu.VMEM((2, DCH), jnp.int32),
            pltpu.VMEM((DCH, DW), jnp.int32),
            pltpu.VMEM((DCH, DW), jnp.int32),
            pltpu.SemaphoreType.DMA,
            pltpu.SemaphoreType.DMA,
        ],
    )(pos_d, x_i32)

    xg_bf = lax.bitcast_convert_type(xg, jnp.bfloat16).reshape(P, D)
    We1b = We1.astype(jnp.bfloat16)
    We2b = We2.astype(jnp.bfloat16)
    be1r = be1.reshape(E, 1, H)
    be2r = be2.reshape(E, 1, D)

    ys = _run_ffn(bexp.reshape(G), xg_bf, We1b, be1r, We2b, be2r)

    y1_bf, y2_bf = pl.kernel(
        _combine_gather_body,
        out_type=(jax.ShapeDtypeStruct((N, D), jnp.float32),
                  jax.ShapeDtypeStruct((N, D), jnp.float32)),
        mesh=plsc.VectorSubcoreMesh(core_axis_name="c",
                                    subcore_axis_name="s"),
        compiler_params=pltpu.CompilerParams(needs_layout_passes=False),
        scratch_types=[
            pltpu.VMEM((2, TCH), jnp.int32),
            pltpu.VMEM((4, CCH, D), jnp.float32),
            pltpu.SemaphoreType.DMA,
        ],
    )(posc, ys)

    BA = 1024
    out = pl.pallas_call(
        _combine_add_body,
        grid=(N // BA,),
        in_specs=[
            pl.BlockSpec((BA, D), lambda g: (g, 0)),
            pl.BlockSpec((BA, D), lambda g: (g, 0)),
            pl.BlockSpec((BA, 1), lambda g: (g, 0)),
            pl.BlockSpec((BA, 1), lambda g: (g, 0)),
        ],
        out_specs=pl.BlockSpec((BA, D), lambda g: (g, 0)),
        out_shape=jax.ShapeDtypeStruct((N, D), jnp.float32),
    )(y1_bf, y2_bf, wsl[:N], wsl[N:])

    return out, aux.reshape(())
